# Initial kernel scaffold; baseline (speedup 1.0000x reference)
import jax
import jax.numpy as jnp
from jax.experimental import pallas as pl


def kernel(x, gamma, beta):
    raise NotImplementedError("write your pallas kernel here")


# speedup vs baseline: 1.0602x; 1.0602x over previous
import functools

import jax
import jax.numpy as jnp
from jax.experimental import pallas as pl
from jax.experimental.pallas import tpu as pltpu

EPS = 1e-5
NEG_SLOPE = 0.01


def _stats_kernel(p_ref, part_ref):
    p = p_ref[...]
    s1 = jnp.sum(jnp.sum(p, axis=2, keepdims=True), axis=0)
    s2 = jnp.sum(jnp.sum(p * p, axis=2, keepdims=True), axis=0)
    part = jnp.concatenate([s1, s2], axis=1)[None]

    j = pl.program_id(1)

    @pl.when(j == 0)
    def _():
        part_ref[...] = part

    @pl.when(j > 0)
    def _():
        part_ref[...] = part_ref[...] + part


def _norm_kernel(p_ref, part_ref, gb_ref, o_ref, *, inv_count):
    part = part_ref[...]
    tot = part[0] + part[1]
    mean = tot[:, 0:1] * inv_count
    ex2 = tot[:, 1:2] * inv_count
    var = jnp.maximum(ex2 - mean * mean, 0.0)
    inv_std = jax.lax.rsqrt(var + EPS)
    scale = gb_ref[:, 0:1] * inv_std
    shift = gb_ref[:, 1:2] - mean * scale
    c = scale.shape[0]
    y = p_ref[...] * scale.reshape(1, c, 1) + shift.reshape(1, c, 1)
    o_ref[...] = jnp.where(y >= 0.0, y, NEG_SLOPE * y)


@jax.jit
def _fused(x, gamma, beta):
    N, C, L = x.shape
    L2 = L // 2
    half = N // 2
    TN = 4
    J = half // TN
    gb = jnp.stack([gamma.astype(jnp.float32), beta.astype(jnp.float32)], axis=1)

    pooled = jnp.max(x.reshape(N, C, L2, 2), axis=-1)

    part = pl.pallas_call(
        _stats_kernel,
        out_shape=jax.ShapeDtypeStruct((2, C, 2), jnp.float32),
        grid=(2, J),
        in_specs=[pl.BlockSpec((TN, C, L2), lambda c, j: (c * J + j, 0, 0))],
        out_specs=pl.BlockSpec((1, C, 2), lambda c, j: (c, 0, 0)),
        compiler_params=pltpu.CompilerParams(
            dimension_semantics=("parallel", "arbitrary"),
            vmem_limit_bytes=64 * 1024 * 1024,
        ),
    )(pooled)

    TN2 = 2
    J2 = half // TN2
    y = pl.pallas_call(
        functools.partial(_norm_kernel, inv_count=1.0 / float(N * L2)),
        out_shape=jax.ShapeDtypeStruct((N, C, L2), x.dtype),
        grid=(2, J2),
        in_specs=[
            pl.BlockSpec((TN2, C, L2), lambda c, j: (c * J2 + j, 0, 0)),
            pl.BlockSpec((2, C, 2), lambda c, j: (0, 0, 0)),
            pl.BlockSpec((C, 2), lambda c, j: (0, 0)),
        ],
        out_specs=pl.BlockSpec((TN2, C, L2), lambda c, j: (c * J2 + j, 0, 0)),
        compiler_params=pltpu.CompilerParams(
            dimension_semantics=("parallel", "parallel"),
            vmem_limit_bytes=64 * 1024 * 1024,
        ),
    )(pooled, part, gb)
    return y


def kernel(x, gamma, beta):
    return _fused(x, gamma, beta)
